# baseline (speedup 1.0000x reference)
import jax
import jax.numpy as jnp
import numpy as np
from jax.experimental import pallas as pl
from jax.experimental.pallas import tpu as pltpu

B = 32
C = 32
F_DIM = 128
H = 32
HH = H * H
E_CNT = C * (C - 1)
JCHUNK = 8
SPB = 4


def _body(x_ref, W_in_ref, b_in_ref, Wst0_ref, Ws0_ref, bs0_ref,
          Wst1_ref, Ws1_ref, bs1_ref, W_out_ref, b_out_ref,
          eye_ref, Emat_ref, Rmat_ref, out_ref):
    f32 = jnp.float32

    def mm(a, b):
        return jax.lax.dot(a, b, preferred_element_type=f32)

    for s in range(SPB):
        _sample(s, x_ref, W_in_ref, b_in_ref, Wst0_ref, Ws0_ref, bs0_ref,
                Wst1_ref, Ws1_ref, bs1_ref, W_out_ref, b_out_ref,
                eye_ref, Emat_ref, Rmat_ref, out_ref, mm)


def _sample(s, x_ref, W_in_ref, b_in_ref, Wst0_ref, Ws0_ref, bs0_ref,
            Wst1_ref, Ws1_ref, bs1_ref, W_out_ref, b_out_ref,
            eye_ref, Emat_ref, Rmat_ref, out_ref, mm):
    f32 = jnp.float32
    xb = x_ref[s]
    h = jnp.maximum(mm(xb, W_in_ref[...]) + b_in_ref[...], 0.0)

    G = jax.lax.dot_general(h, h, (((1,), (1,)), ((), ())),
                            preferred_element_type=f32)
    eye = eye_ref[...]
    n2_col = jnp.sum(G * eye, axis=1, keepdims=True)
    n2_row = jnp.sum(G * eye, axis=0, keepdims=True)
    denom = jnp.maximum(jnp.sqrt(n2_col) * jnp.sqrt(n2_row), 1e-8)
    cos = G / denom
    d2 = jnp.maximum(n2_col + n2_row - 2.0 * G, 0.0)
    dist = jnp.sqrt(d2)
    mean_dist = jnp.sum(dist) * (1.0 / E_CNT)
    distn = dist / (mean_dist + 1e-6)

    lane_iota = jax.lax.broadcasted_iota(jnp.int32, (1, C), 1)
    ones_col = jnp.ones((C, 1), f32)
    zeros_col = jnp.zeros((C, 1), f32)

    def attr3(j):
        onehot = (lane_iota == j).astype(f32)
        cos_col = jnp.sum(cos * onehot, axis=1, keepdims=True)
        dist_col = jnp.sum(distn * onehot, axis=1, keepdims=True)
        return jnp.concatenate([cos_col, dist_col, ones_col, zeros_col],
                               axis=1)

    chunks = [
        jnp.concatenate([attr3(c * JCHUNK + g) for g in range(JCHUNK)], axis=0)
        for c in range(C // JCHUNK)
    ]

    Emat = Emat_ref[...]
    Rmat = Rmat_ref[...]

    def conv(h_in, Wst_ref, Ws_ref, bs_ref):
        Wstack = Wst_ref[...]
        h_exp = mm(h_in, Emat)
        tdiag = jnp.tanh(Wstack[0:1, :] + Wstack[2:3, :])

        s1 = -(h_exp * tdiag)
        for c in range(C // JCHUNK):
            t = jnp.tanh(mm(chunks[c], Wstack))
            for g in range(JCHUNK):
                i = c * JCHUNK + g
                s1 = s1 + t[g * C:(g + 1) * C, :] * h_exp[i:i + 1, :]

        aggr = mm(s1, Rmat) * (1.0 / (C - 1))
        upd = aggr + mm(h_in, Ws_ref[...]) + bs_ref[...]
        return jnp.maximum(upd, 0.0)

    h1 = conv(h, Wst0_ref, Ws0_ref, bs0_ref)
    h2 = conv(h1, Wst1_ref, Ws1_ref, bs1_ref)

    out = mm(h2, W_out_ref[...]) + b_out_ref[...]
    out_ref[s] = xb + out


@jax.jit
def _run(x, W_in, b_in, Wst0, Ws0, bs0, Wst1, Ws1, bs1,
         W_out, b_out, eyeC, Emat, Rmat):
    full = lambda s: pl.BlockSpec(s, lambda b: (0,) * len(s))
    return pl.pallas_call(
        _body,
        grid=(B // SPB,),
        in_specs=[
            pl.BlockSpec((SPB, C, F_DIM), lambda b: (b, 0, 0)),
            full((F_DIM, H)), full((1, H)),
            full((4, HH)),
            full((H, H)), full((1, H)),
            full((4, HH)),
            full((H, H)), full((1, H)),
            full((H, F_DIM)), full((1, F_DIM)),
            full((C, C)),
            full((H, HH)),
            full((HH, H)),
        ],
        out_specs=pl.BlockSpec((SPB, C, F_DIM), lambda b: (b, 0, 0)),
        out_shape=jax.ShapeDtypeStruct((B, C, F_DIM), jnp.float32),
        compiler_params=pltpu.CompilerParams(
            dimension_semantics=("parallel",)),
    )(x, W_in, b_in, Wst0, Ws0, bs0, Wst1, Ws1, bs1,
      W_out, b_out, eyeC, Emat, Rmat)


def kernel(x, W_in, b_in, W_e0, b_e0, W_s0, b_s0, W_e1, b_e1, W_s1, b_s1,
           W_out, b_out):
    eyeC = jnp.asarray(np.eye(C, dtype=np.float32))
    Emat = jnp.asarray(np.kron(np.eye(H), np.ones((1, H))).astype(np.float32))
    Rmat = jnp.asarray(np.tile(np.eye(H), (H, 1)).astype(np.float32))
    zrow = jnp.zeros((1, HH), jnp.float32)
    Wst0 = jnp.concatenate([W_e0, b_e0.reshape(1, HH), zrow], axis=0)
    Wst1 = jnp.concatenate([W_e1, b_e1.reshape(1, HH), zrow], axis=0)
    return _run(x, W_in, b_in.reshape(1, H), Wst0, W_s0, b_s0.reshape(1, H),
                Wst1, W_s1, b_s1.reshape(1, H), W_out, b_out.reshape(1, F_DIM),
                eyeC, Emat, Rmat)


# speedup vs baseline: 1.0463x; 1.0463x over previous
import jax
import jax.numpy as jnp
import numpy as np
from jax.experimental import pallas as pl
from jax.experimental.pallas import tpu as pltpu

B = 32
C = 32
F_DIM = 128
H = 32
HH = H * H
E_CNT = C * (C - 1)
JCHUNK = 8
SPB = 8


def _body(x_ref, W_in_ref, b_in_ref, Wst0_ref, Ws0_ref, bs0_ref,
          Wst1_ref, Ws1_ref, bs1_ref, W_out_ref, b_out_ref,
          eye_ref, Emat_ref, Rmat_ref, out_ref):
    f32 = jnp.float32

    def mm(a, b):
        return jax.lax.dot(a, b, preferred_element_type=f32)

    for s in range(SPB):
        _sample(s, x_ref, W_in_ref, b_in_ref, Wst0_ref, Ws0_ref, bs0_ref,
                Wst1_ref, Ws1_ref, bs1_ref, W_out_ref, b_out_ref,
                eye_ref, Emat_ref, Rmat_ref, out_ref, mm)


def _sample(s, x_ref, W_in_ref, b_in_ref, Wst0_ref, Ws0_ref, bs0_ref,
            Wst1_ref, Ws1_ref, bs1_ref, W_out_ref, b_out_ref,
            eye_ref, Emat_ref, Rmat_ref, out_ref, mm):
    f32 = jnp.float32
    xb = x_ref[s]
    h = jnp.maximum(mm(xb, W_in_ref[...]) + b_in_ref[...], 0.0)

    G = jax.lax.dot_general(h, h, (((1,), (1,)), ((), ())),
                            preferred_element_type=f32)
    eye = eye_ref[...]
    n2_col = jnp.sum(G * eye, axis=1, keepdims=True)
    n2_row = jnp.sum(G * eye, axis=0, keepdims=True)
    denom = jnp.maximum(jnp.sqrt(n2_col) * jnp.sqrt(n2_row), 1e-8)
    cos = G / denom
    d2 = jnp.maximum(n2_col + n2_row - 2.0 * G, 0.0)
    dist = jnp.sqrt(d2)
    mean_dist = jnp.sum(dist) * (1.0 / E_CNT)
    distn = dist / (mean_dist + 1e-6)

    lane_iota = jax.lax.broadcasted_iota(jnp.int32, (1, C), 1)
    ones_col = jnp.ones((C, 1), f32)
    zeros_col = jnp.zeros((C, 1), f32)

    def attr3(j):
        onehot = (lane_iota == j).astype(f32)
        cos_col = jnp.sum(cos * onehot, axis=1, keepdims=True)
        dist_col = jnp.sum(distn * onehot, axis=1, keepdims=True)
        return jnp.concatenate([cos_col, dist_col, ones_col, zeros_col],
                               axis=1)

    chunks = [
        jnp.concatenate([attr3(c * JCHUNK + g) for g in range(JCHUNK)], axis=0)
        for c in range(C // JCHUNK)
    ]

    Emat = Emat_ref[...]
    Rmat = Rmat_ref[...]

    def conv(h_in, Wst_ref, Ws_ref, bs_ref):
        Wstack = Wst_ref[...]
        h_exp = mm(h_in, Emat)
        tdiag = jnp.tanh(Wstack[0:1, :] + Wstack[2:3, :])

        s1 = -(h_exp * tdiag)
        for c in range(C // JCHUNK):
            t = jnp.tanh(mm(chunks[c], Wstack))
            for g in range(JCHUNK):
                i = c * JCHUNK + g
                s1 = s1 + t[g * C:(g + 1) * C, :] * h_exp[i:i + 1, :]

        aggr = mm(s1, Rmat) * (1.0 / (C - 1))
        upd = aggr + mm(h_in, Ws_ref[...]) + bs_ref[...]
        return jnp.maximum(upd, 0.0)

    h1 = conv(h, Wst0_ref, Ws0_ref, bs0_ref)
    h2 = conv(h1, Wst1_ref, Ws1_ref, bs1_ref)

    out = mm(h2, W_out_ref[...]) + b_out_ref[...]
    out_ref[s] = xb + out


@jax.jit
def _run(x, W_in, b_in, Wst0, Ws0, bs0, Wst1, Ws1, bs1,
         W_out, b_out, eyeC, Emat, Rmat):
    full = lambda s: pl.BlockSpec(s, lambda b: (0,) * len(s))
    return pl.pallas_call(
        _body,
        grid=(B // SPB,),
        in_specs=[
            pl.BlockSpec((SPB, C, F_DIM), lambda b: (b, 0, 0)),
            full((F_DIM, H)), full((1, H)),
            full((4, HH)),
            full((H, H)), full((1, H)),
            full((4, HH)),
            full((H, H)), full((1, H)),
            full((H, F_DIM)), full((1, F_DIM)),
            full((C, C)),
            full((H, HH)),
            full((HH, H)),
        ],
        out_specs=pl.BlockSpec((SPB, C, F_DIM), lambda b: (b, 0, 0)),
        out_shape=jax.ShapeDtypeStruct((B, C, F_DIM), jnp.float32),
        compiler_params=pltpu.CompilerParams(
            dimension_semantics=("parallel",)),
    )(x, W_in, b_in, Wst0, Ws0, bs0, Wst1, Ws1, bs1,
      W_out, b_out, eyeC, Emat, Rmat)


def kernel(x, W_in, b_in, W_e0, b_e0, W_s0, b_s0, W_e1, b_e1, W_s1, b_s1,
           W_out, b_out):
    eyeC = jnp.asarray(np.eye(C, dtype=np.float32))
    Emat = jnp.asarray(np.kron(np.eye(H), np.ones((1, H))).astype(np.float32))
    Rmat = jnp.asarray(np.tile(np.eye(H), (H, 1)).astype(np.float32))
    zrow = jnp.zeros((1, HH), jnp.float32)
    Wst0 = jnp.concatenate([W_e0, b_e0.reshape(1, HH), zrow], axis=0)
    Wst1 = jnp.concatenate([W_e1, b_e1.reshape(1, HH), zrow], axis=0)
    return _run(x, W_in, b_in.reshape(1, H), Wst0, W_s0, b_s0.reshape(1, H),
                Wst1, W_s1, b_s1.reshape(1, H), W_out, b_out.reshape(1, F_DIM),
                eyeC, Emat, Rmat)


# speedup vs baseline: 1.0513x; 1.0048x over previous
import jax
import jax.numpy as jnp
import numpy as np
from jax.experimental import pallas as pl
from jax.experimental.pallas import tpu as pltpu

B = 32
C = 32
F_DIM = 128
H = 32
HH = H * H
E_CNT = C * (C - 1)
JCHUNK = 8
SPB = 16


def _body(x_ref, W_in_ref, b_in_ref, Wst0_ref, Ws0_ref, bs0_ref,
          Wst1_ref, Ws1_ref, bs1_ref, W_out_ref, b_out_ref,
          eye_ref, Emat_ref, Rmat_ref, out_ref):
    f32 = jnp.float32

    def mm(a, b):
        return jax.lax.dot(a, b, preferred_element_type=f32)

    for s in range(SPB):
        _sample(s, x_ref, W_in_ref, b_in_ref, Wst0_ref, Ws0_ref, bs0_ref,
                Wst1_ref, Ws1_ref, bs1_ref, W_out_ref, b_out_ref,
                eye_ref, Emat_ref, Rmat_ref, out_ref, mm)


def _sample(s, x_ref, W_in_ref, b_in_ref, Wst0_ref, Ws0_ref, bs0_ref,
            Wst1_ref, Ws1_ref, bs1_ref, W_out_ref, b_out_ref,
            eye_ref, Emat_ref, Rmat_ref, out_ref, mm):
    f32 = jnp.float32
    xb = x_ref[s]
    h = jnp.maximum(mm(xb, W_in_ref[...]) + b_in_ref[...], 0.0)

    G = jax.lax.dot_general(h, h, (((1,), (1,)), ((), ())),
                            preferred_element_type=f32)
    eye = eye_ref[...]
    n2_col = jnp.sum(G * eye, axis=1, keepdims=True)
    n2_row = jnp.sum(G * eye, axis=0, keepdims=True)
    denom = jnp.maximum(jnp.sqrt(n2_col) * jnp.sqrt(n2_row), 1e-8)
    cos = G / denom
    d2 = jnp.maximum(n2_col + n2_row - 2.0 * G, 0.0)
    dist = jnp.sqrt(d2)
    mean_dist = jnp.sum(dist) * (1.0 / E_CNT)
    distn = dist / (mean_dist + 1e-6)

    lane_iota = jax.lax.broadcasted_iota(jnp.int32, (1, C), 1)
    ones_col = jnp.ones((C, 1), f32)
    zeros_col = jnp.zeros((C, 1), f32)

    def attr3(j):
        onehot = (lane_iota == j).astype(f32)
        cos_col = jnp.sum(cos * onehot, axis=1, keepdims=True)
        dist_col = jnp.sum(distn * onehot, axis=1, keepdims=True)
        return jnp.concatenate([cos_col, dist_col, ones_col, zeros_col],
                               axis=1)

    chunks = [
        jnp.concatenate([attr3(c * JCHUNK + g) for g in range(JCHUNK)], axis=0)
        for c in range(C // JCHUNK)
    ]

    Emat = Emat_ref[...]
    Rmat = Rmat_ref[...]

    def conv(h_in, Wst_ref, Ws_ref, bs_ref):
        Wstack = Wst_ref[...]
        h_exp = mm(h_in, Emat)
        tdiag = jnp.tanh(Wstack[0:1, :] + Wstack[2:3, :])

        s1 = -(h_exp * tdiag)
        for c in range(C // JCHUNK):
            t = jnp.tanh(mm(chunks[c], Wstack))
            for g in range(JCHUNK):
                i = c * JCHUNK + g
                s1 = s1 + t[g * C:(g + 1) * C, :] * h_exp[i:i + 1, :]

        aggr = mm(s1, Rmat) * (1.0 / (C - 1))
        upd = aggr + mm(h_in, Ws_ref[...]) + bs_ref[...]
        return jnp.maximum(upd, 0.0)

    h1 = conv(h, Wst0_ref, Ws0_ref, bs0_ref)
    h2 = conv(h1, Wst1_ref, Ws1_ref, bs1_ref)

    out = mm(h2, W_out_ref[...]) + b_out_ref[...]
    out_ref[s] = xb + out


@jax.jit
def _run(x, W_in, b_in, Wst0, Ws0, bs0, Wst1, Ws1, bs1,
         W_out, b_out, eyeC, Emat, Rmat):
    full = lambda s: pl.BlockSpec(s, lambda b: (0,) * len(s))
    return pl.pallas_call(
        _body,
        grid=(B // SPB,),
        in_specs=[
            pl.BlockSpec((SPB, C, F_DIM), lambda b: (b, 0, 0)),
            full((F_DIM, H)), full((1, H)),
            full((4, HH)),
            full((H, H)), full((1, H)),
            full((4, HH)),
            full((H, H)), full((1, H)),
            full((H, F_DIM)), full((1, F_DIM)),
            full((C, C)),
            full((H, HH)),
            full((HH, H)),
        ],
        out_specs=pl.BlockSpec((SPB, C, F_DIM), lambda b: (b, 0, 0)),
        out_shape=jax.ShapeDtypeStruct((B, C, F_DIM), jnp.float32),
        compiler_params=pltpu.CompilerParams(
            dimension_semantics=("parallel",)),
    )(x, W_in, b_in, Wst0, Ws0, bs0, Wst1, Ws1, bs1,
      W_out, b_out, eyeC, Emat, Rmat)


def kernel(x, W_in, b_in, W_e0, b_e0, W_s0, b_s0, W_e1, b_e1, W_s1, b_s1,
           W_out, b_out):
    eyeC = jnp.asarray(np.eye(C, dtype=np.float32))
    Emat = jnp.asarray(np.kron(np.eye(H), np.ones((1, H))).astype(np.float32))
    Rmat = jnp.asarray(np.tile(np.eye(H), (H, 1)).astype(np.float32))
    zrow = jnp.zeros((1, HH), jnp.float32)
    Wst0 = jnp.concatenate([W_e0, b_e0.reshape(1, HH), zrow], axis=0)
    Wst1 = jnp.concatenate([W_e1, b_e1.reshape(1, HH), zrow], axis=0)
    return _run(x, W_in, b_in.reshape(1, H), Wst0, W_s0, b_s0.reshape(1, H),
                Wst1, W_s1, b_s1.reshape(1, H), W_out, b_out.reshape(1, F_DIM),
                eyeC, Emat, Rmat)
